# baseline (speedup 1.0000x reference)
import jax
import jax.numpy as jnp
from jax.experimental import pallas as pl
from jax.experimental.pallas import tpu as pltpu

N = 10000
NFEAT = 128
NHID = 16
NCLASS = 8

TI = 400
NI = N // TI
D = 3
NF = NI - D


def _log_softmax(z):
    m = jnp.max(z, axis=1, keepdims=True)
    return z - (jnp.log(jnp.sum(jnp.exp(z - m), axis=1, keepdims=True)) + m)


def _body(adj_hbm, x_hbm, w1_ref, w2_ref, b1_ref, b2_ref, o_hbm,
          s1_ref, s2_ref, x_ref, st0, st1, st2,
          buf0, buf1, buf2, sem0, sem1, sem2, xsem, os0, os1, os2):
    bufs = (buf0, buf1, buf2)
    sems = (sem0, sem1, sem2)
    stages = (st0, st1, st2)
    osems = (os0, os1, os2)

    def copy(block_start, slot):
        return pltpu.make_async_copy(
            adj_hbm.at[pl.ds(block_start * TI, TI), :], bufs[slot],
            sems[slot])

    def out_copy(block_idx, slot):
        return pltpu.make_async_copy(
            stages[slot], o_hbm.at[pl.ds(block_idx * TI, TI), :],
            osems[slot])

    for d in range(D):
        copy(d, d).start()
    xcp = pltpu.make_async_copy(x_hbm, x_ref, xsem)
    xcp.start()
    xcp.wait()
    s1_ref[...] = jnp.dot(x_ref[...], w1_ref[...],
                          preferred_element_type=jnp.float32).astype(
                              jnp.bfloat16)

    def b_step(i, slot):
        copy(i, slot).wait()
        h = jnp.maximum(jnp.dot(bufs[slot][...], s1_ref[...],
                                preferred_element_type=jnp.float32)
                        + b1_ref[...], 0.0)
        s2_ref[pl.ds(i * TI, TI), :] = jnp.dot(
            h, w2_ref[...], preferred_element_type=jnp.float32)

        @pl.when(i + D < NI)
        def _():
            copy(i + D, slot).start()

    def b_loop(k, carry):
        for d in range(D):
            b_step(k * D + d, d)
        return carry

    nb_main = (NI // D) * D
    jax.lax.fori_loop(0, NI // D, b_loop, 0)
    for i in range(nb_main, NI):
        b_step(i, i % D)

    def emit(block_idx, slot, first=False):
        z = jnp.dot(bufs[slot][...], s2_ref[...],
                    preferred_element_type=jnp.float32) + b2_ref[...]
        if not first:
            out_copy(block_idx, slot).wait()
        stages[slot][...] = _log_softmax(z)
        out_copy(block_idx, slot).start()

    for t in range(D):
        blk_id = NI - 1 - t
        slot = blk_id % D
        emit(blk_id, slot, first=True)
        if t < NF:
            copy(t, slot).start()

    def f_step(b, slot):
        copy(b, slot).wait()
        emit(b, slot)

        @pl.when(b + D < NF)
        def _():
            copy(b + D, slot).start()

    def f_loop(k, carry):
        for d in range(D):
            f_step(k * D + d, (NI - 1 - d) % D)
        return carry

    nf_main = (NF // D) * D
    jax.lax.fori_loop(0, NF // D, f_loop, 0)
    for j in range(nf_main, NF):
        f_step(j, (NI - 1 - j) % D)

    for d in range(D):
        last = NF - 1 - d
        out_copy(last, (NI - 1 - last) % D).wait()


def kernel(x, adj, W1, b1, W2, b2):
    return pl.pallas_call(
        _body,
        in_specs=[
            pl.BlockSpec(memory_space=pl.ANY),
            pl.BlockSpec(memory_space=pl.ANY),
            pl.BlockSpec(memory_space=pltpu.MemorySpace.VMEM),
            pl.BlockSpec(memory_space=pltpu.MemorySpace.VMEM),
            pl.BlockSpec(memory_space=pltpu.MemorySpace.VMEM),
            pl.BlockSpec(memory_space=pltpu.MemorySpace.VMEM),
        ],
        out_specs=pl.BlockSpec(memory_space=pl.ANY),
        out_shape=jax.ShapeDtypeStruct((N, NCLASS), jnp.float32),
        scratch_shapes=[
            pltpu.VMEM((N, NHID), jnp.bfloat16),
            pltpu.VMEM((N, NCLASS), jnp.float32),
            pltpu.VMEM((N, NFEAT), jnp.float32),
            pltpu.VMEM((TI, NCLASS), jnp.float32),
            pltpu.VMEM((TI, NCLASS), jnp.float32),
            pltpu.VMEM((TI, NCLASS), jnp.float32),
            pltpu.VMEM((TI, N), jnp.float32),
            pltpu.VMEM((TI, N), jnp.float32),
            pltpu.VMEM((TI, N), jnp.float32),
            pltpu.SemaphoreType.DMA,
            pltpu.SemaphoreType.DMA,
            pltpu.SemaphoreType.DMA,
            pltpu.SemaphoreType.DMA,
            pltpu.SemaphoreType.DMA,
            pltpu.SemaphoreType.DMA,
            pltpu.SemaphoreType.DMA,
        ],
        compiler_params=pltpu.CompilerParams(
            vmem_limit_bytes=100 * 1024 * 1024),
    )(adj, x, W1, W2, b1.reshape(1, NHID), b2.reshape(1, NCLASS))


# speedup vs baseline: 1.0427x; 1.0427x over previous
import jax
import jax.numpy as jnp
from jax.experimental import pallas as pl
from jax.experimental.pallas import tpu as pltpu

N = 10000
NFEAT = 128
NHID = 16
NCLASS = 8

TI = 200
NI = N // TI
D = 3
NCB = 4
TC = 208
NF = NI - D - NCB


def _log_softmax(z):
    m = jnp.max(z, axis=1, keepdims=True)
    return z - (jnp.log(jnp.sum(jnp.exp(z - m), axis=1, keepdims=True)) + m)


XH = 5008


def _body(adj_hbm, x_hbm, w1_ref, w2_ref, b1_ref, b2_ref, o_ref,
          s1_ref, s2_ref, cache_ref, xh_ref, buf0, buf1, buf2,
          sem0, sem1, sem2, xsem):
    bufs = (buf0, buf1, buf2)
    sems = (sem0, sem1, sem2)

    def copy(block_start, slot):
        return pltpu.make_async_copy(
            adj_hbm.at[pl.ds(block_start * TI, TI), :], bufs[slot],
            sems[slot])

    for d in range(D):
        copy(d, d).start()
    xc1 = pltpu.make_async_copy(x_hbm.at[pl.ds(0, XH), :], xh_ref, xsem)
    xc1.start()
    xc1.wait()
    s1_ref[pl.ds(0, XH), :] = jnp.dot(
        xh_ref[...], w1_ref[...],
        preferred_element_type=jnp.float32).astype(jnp.bfloat16)
    xc2 = pltpu.make_async_copy(x_hbm.at[pl.ds(XH, N - XH), :],
                                xh_ref.at[pl.ds(0, N - XH), :], xsem)
    xc2.start()
    xc2.wait()
    s1_ref[pl.ds(XH, N - XH), :] = jnp.dot(
        xh_ref[pl.ds(0, N - XH), :], w1_ref[...],
        preferred_element_type=jnp.float32).astype(jnp.bfloat16)

    def b_step(i, slot):
        copy(i, slot).wait()
        blk = bufs[slot][...]
        h = jnp.maximum(jnp.dot(blk, s1_ref[...],
                                preferred_element_type=jnp.float32)
                        + b1_ref[...], 0.0)
        s2_ref[pl.ds(i * TI, TI), :] = jnp.dot(
            h, w2_ref[...], preferred_element_type=jnp.float32)

        @pl.when(i < NCB)
        def _():
            cache_ref[pl.ds(i * TC, TI), :] = blk.astype(jnp.bfloat16)

        @pl.when(i + D < NI)
        def _():
            copy(i + D, slot).start()

    def b_loop(k, carry):
        for d in range(D):
            b_step(k * D + d, d)
        return carry

    nb_main = (NI // D) * D
    jax.lax.fori_loop(0, NI // D, b_loop, 0)
    for i in range(nb_main, NI):
        b_step(i, i % D)

    def emit(block_idx, src):
        z = jnp.dot(src, s2_ref[...],
                    preferred_element_type=jnp.float32) + b2_ref[...]
        o_ref[pl.ds(block_idx * TI, TI), :] = _log_softmax(z)

    for t in range(D):
        blk_id = NI - 1 - t
        slot = blk_id % D
        emit(blk_id, bufs[slot][...])
        if t < NF:
            copy(NCB + t, slot).start()

    def c_cache(m, carry):
        z = jnp.dot(cache_ref[pl.ds(m * TC, TI), :],
                    s2_ref[...].astype(jnp.bfloat16),
                    preferred_element_type=jnp.float32) + b2_ref[...]
        o_ref[pl.ds(m * TI, TI), :] = _log_softmax(z)
        return carry

    jax.lax.fori_loop(0, NCB, c_cache, 0)

    def f_step(b, slot):
        copy(b, slot).wait()
        emit(b, bufs[slot][...])

        @pl.when(b + D < NCB + NF)
        def _():
            copy(b + D, slot).start()

    def f_loop(k, carry):
        for d in range(D):
            b = NCB + k * D + d
            f_step(b, (NI - 1 - d) % D)
        return carry

    nf_main = (NF // D) * D
    jax.lax.fori_loop(0, NF // D, f_loop, 0)
    for j in range(nf_main, NF):
        f_step(NCB + j, (NI - 1 - j) % D)


def kernel(x, adj, W1, b1, W2, b2):
    return pl.pallas_call(
        _body,
        in_specs=[
            pl.BlockSpec(memory_space=pl.ANY),
            pl.BlockSpec(memory_space=pl.ANY),
            pl.BlockSpec(memory_space=pltpu.MemorySpace.VMEM),
            pl.BlockSpec(memory_space=pltpu.MemorySpace.VMEM),
            pl.BlockSpec(memory_space=pltpu.MemorySpace.VMEM),
            pl.BlockSpec(memory_space=pltpu.MemorySpace.VMEM),
        ],
        out_specs=pl.BlockSpec(memory_space=pltpu.MemorySpace.VMEM),
        out_shape=jax.ShapeDtypeStruct((N, NCLASS), jnp.float32),
        scratch_shapes=[
            pltpu.VMEM((N, NHID), jnp.bfloat16),
            pltpu.VMEM((N, NCLASS), jnp.float32),
            pltpu.VMEM((NCB * TC, N), jnp.bfloat16),
            pltpu.VMEM((XH, NFEAT), jnp.float32),
            pltpu.VMEM((TI, N), jnp.float32),
            pltpu.VMEM((TI, N), jnp.float32),
            pltpu.VMEM((TI, N), jnp.float32),
            pltpu.SemaphoreType.DMA,
            pltpu.SemaphoreType.DMA,
            pltpu.SemaphoreType.DMA,
            pltpu.SemaphoreType.DMA,
        ],
        compiler_params=pltpu.CompilerParams(
            vmem_limit_bytes=100 * 1024 * 1024),
    )(adj, x, W1, W2, b1.reshape(1, NHID), b2.reshape(1, NCLASS))


# speedup vs baseline: 1.0605x; 1.0171x over previous
import jax
import jax.numpy as jnp
from jax.experimental import pallas as pl
from jax.experimental.pallas import tpu as pltpu

N = 10000
NFEAT = 128
NHID = 16
NCLASS = 8

TI = 200
NI = N // TI
D = 3
NCB = 3
TC = 208
NF = NI - D - NCB


def _log_softmax(z):
    m = jnp.max(z, axis=1, keepdims=True)
    return z - (jnp.log(jnp.sum(jnp.exp(z - m), axis=1, keepdims=True)) + m)


def _body(adj_hbm, x_ref, w1_ref, w2_ref, b1_ref, b2_ref, o_ref,
          s1_ref, s2_ref, cache_ref, buf0, buf1, buf2, sem0, sem1, sem2):
    bufs = (buf0, buf1, buf2)
    sems = (sem0, sem1, sem2)

    def copy(block_start, slot):
        return pltpu.make_async_copy(
            adj_hbm.at[pl.ds(block_start * TI, TI), :], bufs[slot],
            sems[slot])

    for d in range(D):
        copy(d, d).start()
    s1_ref[...] = jnp.dot(x_ref[...], w1_ref[...],
                          preferred_element_type=jnp.float32).astype(
                              jnp.bfloat16)

    def b_step(i, slot):
        copy(i, slot).wait()
        blk = bufs[slot][...]
        h = jnp.maximum(jnp.dot(blk, s1_ref[...],
                                preferred_element_type=jnp.float32)
                        + b1_ref[...], 0.0)
        s2_ref[pl.ds(i * TI, TI), :] = jnp.dot(
            h, w2_ref[...], preferred_element_type=jnp.float32)

        @pl.when(i < NCB)
        def _():
            cache_ref[pl.ds(i * TC, TI), :] = blk.astype(jnp.bfloat16)

        @pl.when(i + D < NI)
        def _():
            copy(i + D, slot).start()

    def b_loop(k, carry):
        for d in range(D):
            b_step(k * D + d, d)
        return carry

    nb_main = (NI // D) * D
    jax.lax.fori_loop(0, NI // D, b_loop, 0)
    for i in range(nb_main, NI):
        b_step(i, i % D)

    def emit(block_idx, src):
        z = jnp.dot(src, s2_ref[...],
                    preferred_element_type=jnp.float32) + b2_ref[...]
        o_ref[pl.ds(block_idx * TI, TI), :] = _log_softmax(z)

    for t in range(D):
        blk_id = NI - 1 - t
        slot = blk_id % D
        emit(blk_id, bufs[slot][...])
        if t < NF:
            copy(NCB + t, slot).start()

    def c_cache(m, carry):
        emit(m, cache_ref[pl.ds(m * TC, TI), :])
        return carry

    jax.lax.fori_loop(0, NCB, c_cache, 0)

    def f_step(b, slot):
        copy(b, slot).wait()
        emit(b, bufs[slot][...])

        @pl.when(b + D < NCB + NF)
        def _():
            copy(b + D, slot).start()

    def f_loop(k, carry):
        for d in range(D):
            b = NCB + k * D + d
            f_step(b, (NI - 1 - d) % D)
        return carry

    nf_main = (NF // D) * D
    jax.lax.fori_loop(0, NF // D, f_loop, 0)
    for j in range(nf_main, NF):
        f_step(NCB + j, (NI - 1 - j) % D)


def kernel(x, adj, W1, b1, W2, b2):
    return pl.pallas_call(
        _body,
        in_specs=[
            pl.BlockSpec(memory_space=pl.ANY),
            pl.BlockSpec(memory_space=pltpu.MemorySpace.VMEM),
            pl.BlockSpec(memory_space=pltpu.MemorySpace.VMEM),
            pl.BlockSpec(memory_space=pltpu.MemorySpace.VMEM),
            pl.BlockSpec(memory_space=pltpu.MemorySpace.VMEM),
            pl.BlockSpec(memory_space=pltpu.MemorySpace.VMEM),
        ],
        out_specs=pl.BlockSpec(memory_space=pltpu.MemorySpace.VMEM),
        out_shape=jax.ShapeDtypeStruct((N, NCLASS), jnp.float32),
        scratch_shapes=[
            pltpu.VMEM((N, NHID), jnp.bfloat16),
            pltpu.VMEM((N, NCLASS), jnp.float32),
            pltpu.VMEM((NCB * TC, N), jnp.bfloat16),
            pltpu.VMEM((TI, N), jnp.float32),
            pltpu.VMEM((TI, N), jnp.float32),
            pltpu.VMEM((TI, N), jnp.float32),
            pltpu.SemaphoreType.DMA,
            pltpu.SemaphoreType.DMA,
            pltpu.SemaphoreType.DMA,
        ],
        compiler_params=pltpu.CompilerParams(
            vmem_limit_bytes=100 * 1024 * 1024),
    )(adj, x, W1, W2, b1.reshape(1, NHID), b2.reshape(1, NCLASS))


# speedup vs baseline: 1.0694x; 1.0083x over previous
import jax
import jax.numpy as jnp
from jax.experimental import pallas as pl
from jax.experimental.pallas import tpu as pltpu

N = 10000
NFEAT = 128
NHID = 16
NCLASS = 8

TI = 200
NI = N // TI
D = 3
NCB = 4
TC = 208
NF = NI - D - NCB


def _log_softmax(z):
    m = jnp.max(z, axis=1, keepdims=True)
    return z - (jnp.log(jnp.sum(jnp.exp(z - m), axis=1, keepdims=True)) + m)


def _body(adj_hbm, x_ref, w1_ref, w2_ref, b1_ref, b2_ref, o_ref,
          s1_ref, s2_ref, cache_ref, buf0, buf1, buf2, sem0, sem1, sem2):
    bufs = (buf0, buf1, buf2)
    sems = (sem0, sem1, sem2)

    def copy(block_start, slot):
        return pltpu.make_async_copy(
            adj_hbm.at[pl.ds(block_start * TI, TI), :], bufs[slot],
            sems[slot])

    for d in range(D):
        copy(d, d).start()
    s1_ref[...] = jnp.dot(x_ref[...], w1_ref[...],
                          preferred_element_type=jnp.float32).astype(
                              jnp.bfloat16)

    def b_step(i, slot):
        copy(i, slot).wait()
        h = jnp.maximum(jnp.dot(bufs[slot][...], s1_ref[...],
                                preferred_element_type=jnp.float32)
                        + b1_ref[...], 0.0)
        s2_ref[pl.ds(i * TI, TI), :] = jnp.dot(
            h, w2_ref[...], preferred_element_type=jnp.float32)

        @pl.when(i < NCB)
        def _():
            cache_ref[pl.ds(i * TC, TI), :] = bufs[slot][...].astype(
                jnp.bfloat16)

        @pl.when(i + D < NI)
        def _():
            copy(i + D, slot).start()

    def b_loop(k, carry):
        for d in range(D):
            b_step(k * D + d, d)
        return carry

    nb_main = (NI // D) * D
    jax.lax.fori_loop(0, NI // D, b_loop, 0)
    for i in range(nb_main, NI):
        b_step(i, i % D)

    def emit(block_idx, src):
        z = jnp.dot(src, s2_ref[...],
                    preferred_element_type=jnp.float32) + b2_ref[...]
        o_ref[pl.ds(block_idx * TI, TI), :] = _log_softmax(z)

    for t in range(D):
        blk_id = NI - 1 - t
        slot = blk_id % D
        emit(blk_id, bufs[slot][...])
        if t < NF:
            copy(NCB + t, slot).start()

    def c_cache(m, carry):
        emit(m, cache_ref[pl.ds(m * TC, TI), :])
        return carry

    jax.lax.fori_loop(0, NCB, c_cache, 0)

    def f_step(b, slot):
        copy(b, slot).wait()
        emit(b, bufs[slot][...])

        @pl.when(b + D < NCB + NF)
        def _():
            copy(b + D, slot).start()

    def f_loop(k, carry):
        for d in range(D):
            b = NCB + k * D + d
            f_step(b, (NI - 1 - d) % D)
        return carry

    nf_main = (NF // D) * D
    jax.lax.fori_loop(0, NF // D, f_loop, 0)
    for j in range(nf_main, NF):
        f_step(NCB + j, (NI - 1 - j) % D)


def kernel(x, adj, W1, b1, W2, b2):
    return pl.pallas_call(
        _body,
        in_specs=[
            pl.BlockSpec(memory_space=pl.ANY),
            pl.BlockSpec(memory_space=pltpu.MemorySpace.VMEM),
            pl.BlockSpec(memory_space=pltpu.MemorySpace.VMEM),
            pl.BlockSpec(memory_space=pltpu.MemorySpace.VMEM),
            pl.BlockSpec(memory_space=pltpu.MemorySpace.VMEM),
            pl.BlockSpec(memory_space=pltpu.MemorySpace.VMEM),
        ],
        out_specs=pl.BlockSpec(memory_space=pltpu.MemorySpace.VMEM),
        out_shape=jax.ShapeDtypeStruct((N, NCLASS), jnp.float32),
        scratch_shapes=[
            pltpu.VMEM((N, NHID), jnp.bfloat16),
            pltpu.VMEM((N, NCLASS), jnp.float32),
            pltpu.VMEM((NCB * TC, N), jnp.bfloat16),
            pltpu.VMEM((TI, N), jnp.float32),
            pltpu.VMEM((TI, N), jnp.float32),
            pltpu.VMEM((TI, N), jnp.float32),
            pltpu.SemaphoreType.DMA,
            pltpu.SemaphoreType.DMA,
            pltpu.SemaphoreType.DMA,
        ],
        compiler_params=pltpu.CompilerParams(
            vmem_limit_bytes=100 * 1024 * 1024),
    )(adj, x, W1, W2, b1.reshape(1, NHID), b2.reshape(1, NCLASS))


# speedup vs baseline: 1.0713x; 1.0018x over previous
import jax
import jax.numpy as jnp
from jax.experimental import pallas as pl
from jax.experimental.pallas import tpu as pltpu

N = 10000
NFEAT = 128
NHID = 16
NCLASS = 8

TI = 200
NI = N // TI
D = 3
NCB = 5
TC = 208
NF = NI - D - NCB


def _log_softmax(z):
    m = jnp.max(z, axis=1, keepdims=True)
    return z - (jnp.log(jnp.sum(jnp.exp(z - m), axis=1, keepdims=True)) + m)


def _body(adj_hbm, x_ref, w1_ref, w2_ref, b1_ref, b2_ref, o_ref,
          s1_ref, s2_ref, cache_ref, buf0, buf1, buf2, sem0, sem1, sem2):
    bufs = (buf0, buf1, buf2)
    sems = (sem0, sem1, sem2)

    def copy(block_start, slot):
        return pltpu.make_async_copy(
            adj_hbm.at[pl.ds(block_start * TI, TI), :], bufs[slot],
            sems[slot])

    for d in range(D):
        copy(d, d).start()
    s1_ref[...] = jnp.dot(x_ref[...], w1_ref[...],
                          preferred_element_type=jnp.float32).astype(
                              jnp.bfloat16)

    def b_step(i, slot):
        copy(i, slot).wait()
        h = jnp.maximum(jnp.dot(bufs[slot][...], s1_ref[...],
                                preferred_element_type=jnp.float32)
                        + b1_ref[...], 0.0)
        s2_ref[pl.ds(i * TI, TI), :] = jnp.dot(
            h, w2_ref[...], preferred_element_type=jnp.float32)

        @pl.when(i < NCB)
        def _():
            cache_ref[pl.ds(i * TC, TI), :] = bufs[slot][...].astype(
                jnp.bfloat16)

        @pl.when(i + D < NI)
        def _():
            copy(i + D, slot).start()

    def b_loop(k, carry):
        for d in range(D):
            b_step(k * D + d, d)
        return carry

    nb_main = (NI // D) * D
    jax.lax.fori_loop(0, NI // D, b_loop, 0)
    for i in range(nb_main, NI):
        b_step(i, i % D)

    def emit(block_idx, src):
        z = jnp.dot(src, s2_ref[...],
                    preferred_element_type=jnp.float32) + b2_ref[...]
        o_ref[pl.ds(block_idx * TI, TI), :] = _log_softmax(z)

    for t in range(D):
        blk_id = NI - 1 - t
        slot = blk_id % D
        emit(blk_id, bufs[slot][...])
        if t < NF:
            copy(NCB + t, slot).start()

    def c_cache(m, carry):
        emit(m, cache_ref[pl.ds(m * TC, TI), :])
        return carry

    jax.lax.fori_loop(0, NCB, c_cache, 0)

    def f_step(b, slot):
        copy(b, slot).wait()
        emit(b, bufs[slot][...])

        @pl.when(b + D < NCB + NF)
        def _():
            copy(b + D, slot).start()

    def f_loop(k, carry):
        for d in range(D):
            b = NCB + k * D + d
            f_step(b, (NI - 1 - d) % D)
        return carry

    nf_main = (NF // D) * D
    jax.lax.fori_loop(0, NF // D, f_loop, 0)
    for j in range(nf_main, NF):
        f_step(NCB + j, (NI - 1 - j) % D)


def kernel(x, adj, W1, b1, W2, b2):
    return pl.pallas_call(
        _body,
        in_specs=[
            pl.BlockSpec(memory_space=pl.ANY),
            pl.BlockSpec(memory_space=pltpu.MemorySpace.VMEM),
            pl.BlockSpec(memory_space=pltpu.MemorySpace.VMEM),
            pl.BlockSpec(memory_space=pltpu.MemorySpace.VMEM),
            pl.BlockSpec(memory_space=pltpu.MemorySpace.VMEM),
            pl.BlockSpec(memory_space=pltpu.MemorySpace.VMEM),
        ],
        out_specs=pl.BlockSpec(memory_space=pltpu.MemorySpace.VMEM),
        out_shape=jax.ShapeDtypeStruct((N, NCLASS), jnp.float32),
        scratch_shapes=[
            pltpu.VMEM((N, NHID), jnp.bfloat16),
            pltpu.VMEM((N, NCLASS), jnp.float32),
            pltpu.VMEM((NCB * TC, N), jnp.bfloat16),
            pltpu.VMEM((TI, N), jnp.float32),
            pltpu.VMEM((TI, N), jnp.float32),
            pltpu.VMEM((TI, N), jnp.float32),
            pltpu.SemaphoreType.DMA,
            pltpu.SemaphoreType.DMA,
            pltpu.SemaphoreType.DMA,
        ],
        compiler_params=pltpu.CompilerParams(
            vmem_limit_bytes=100 * 1024 * 1024),
    )(adj, x, W1, W2, b1.reshape(1, NHID), b2.reshape(1, NCLASS))
